# Initial kernel scaffold; baseline (speedup 1.0000x reference)
import jax
import jax.numpy as jnp
from jax.experimental import pallas as pl


def kernel(x, expert_indices, expert_weights, gate_w, gate_b):
    raise NotImplementedError("write your pallas kernel here")


# speedup vs baseline: 1.5602x; 1.5602x over previous
import functools

import jax
import jax.numpy as jnp
from jax.experimental import pallas as pl
from jax.experimental.pallas import tpu as pltpu

T = 64
D = 768
H = 512
H2 = 2 * H
E = 64


def _moe_body(eid_ref, order_ref, x_ref, gw_ref, gb_ref, w_ref, out_ref):
    i = pl.program_id(0)
    e = eid_ref[i]
    row = x_ref[0]
    logits = jnp.dot(row, gw_ref[...], preferred_element_type=jnp.float32)
    logits = logits + gb_ref[...]
    m = jnp.max(logits)
    p = jnp.exp(logits - m)
    probs = p / jnp.sum(p)
    sel = jax.lax.broadcasted_iota(jnp.int32, (1, E), 1) == e
    scale = jnp.sum(jnp.where(sel, probs, 0.0))
    proj = jnp.dot(row, w_ref[0], preferred_element_type=jnp.float32)
    a = proj[:, :H]
    b = proj[:, H:]
    out_ref[0] = jax.lax.logistic(a) * a * b * scale


@jax.jit
def _moe_call(sorted_eid, order, x3, gw, gb2, ew):
    grid_spec = pltpu.PrefetchScalarGridSpec(
        num_scalar_prefetch=2,
        grid=(T,),
        in_specs=[
            pl.BlockSpec((1, 1, D), lambda i, eid, od: (od[i], 0, 0)),
            pl.BlockSpec((D, E), lambda i, eid, od: (0, 0)),
            pl.BlockSpec((1, E), lambda i, eid, od: (0, 0)),
            pl.BlockSpec((1, D, H2), lambda i, eid, od: (eid[i], 0, 0)),
        ],
        out_specs=pl.BlockSpec((1, 1, H), lambda i, eid, od: (od[i], 0, 0)),
    )
    out = pl.pallas_call(
        _moe_body,
        grid_spec=grid_spec,
        out_shape=jax.ShapeDtypeStruct((T, 1, H), jnp.float32),
        compiler_params=pltpu.CompilerParams(
            dimension_semantics=("arbitrary",),
        ),
    )(sorted_eid, order, x3, gw, gb2, ew)
    return out.reshape(T, H)


def kernel(x, expert_indices, expert_weights, gate_w, gate_b):
    order = jnp.argsort(expert_indices)
    sorted_eid = jnp.take(expert_indices, order)
    x3 = x.reshape(T, 1, D)
    gb2 = gate_b.reshape(1, E)
    return _moe_call(sorted_eid, order, x3, gate_w, gb2, expert_weights)


# speedup vs baseline: 1.5756x; 1.0098x over previous
import functools

import jax
import jax.numpy as jnp
from jax.experimental import pallas as pl
from jax.experimental.pallas import tpu as pltpu

T = 64
D = 768
H = 512
H2 = 2 * H
E = 64

NCHUNK = 4
CW = H2 // NCHUNK


def _moe_body(eid_ref, order_ref, x_ref, gw_ref, gb_ref, *rest):
    w_refs = rest[:NCHUNK]
    out_ref = rest[NCHUNK]
    i = pl.program_id(0)
    e = eid_ref[i]
    row = x_ref[0]
    logits = jnp.dot(row, gw_ref[...], preferred_element_type=jnp.float32)
    logits = logits + gb_ref[...]
    m = jnp.max(logits)
    p = jnp.exp(logits - m)
    probs = p / jnp.sum(p)
    sel = jax.lax.broadcasted_iota(jnp.int32, (1, E), 1) == e
    scale = jnp.sum(jnp.where(sel, probs, 0.0))
    proj = jnp.concatenate(
        [jnp.dot(row, w_ref[0], preferred_element_type=jnp.float32)
         for w_ref in w_refs],
        axis=-1,
    )
    a = proj[:, :H]
    b = proj[:, H:]
    out_ref[0] = jax.lax.logistic(a) * a * b * scale


def _w_spec(c):
    return pl.BlockSpec((1, D, CW), lambda i, eid, od: (eid[i], 0, c))


@jax.jit
def _moe_call(sorted_eid, order, x3, gw, gb2, ew):
    grid_spec = pltpu.PrefetchScalarGridSpec(
        num_scalar_prefetch=2,
        grid=(T,),
        in_specs=[
            pl.BlockSpec((1, 1, D), lambda i, eid, od: (od[i], 0, 0)),
            pl.BlockSpec((D, E), lambda i, eid, od: (0, 0)),
            pl.BlockSpec((1, E), lambda i, eid, od: (0, 0)),
        ] + [_w_spec(c) for c in range(NCHUNK)],
        out_specs=pl.BlockSpec((1, 1, H), lambda i, eid, od: (od[i], 0, 0)),
    )
    out = pl.pallas_call(
        _moe_body,
        grid_spec=grid_spec,
        out_shape=jax.ShapeDtypeStruct((T, 1, H), jnp.float32),
        compiler_params=pltpu.CompilerParams(
            dimension_semantics=("arbitrary",),
        ),
    )(sorted_eid, order, x3, gw, gb2, *([ew] * NCHUNK))
    return out.reshape(T, H)


def kernel(x, expert_indices, expert_weights, gate_w, gate_b):
    order = jnp.argsort(expert_indices)
    sorted_eid = jnp.take(expert_indices, order)
    x3 = x.reshape(T, 1, D)
    gb2 = gate_b.reshape(1, E)
    return _moe_call(sorted_eid, order, x3, gate_w, gb2, expert_weights)
